# baseline (speedup 1.0000x reference)
import functools

import jax
import jax.numpy as jnp
from jax import lax
from jax.experimental import pallas as pl
from jax.experimental.pallas import tpu as pltpu
from jax.experimental.pallas import tpu_sc as plsc

N = 10000
E = 320000
F_IN = 128
H = 512
G = 64

NC = 2
NS = 16
CW = 128
GC = 8
NCH = 2560
EP = NCH * CW
NG2 = NCH // NS // GC
NG1 = NCH // (NC * NS) // GC
RQ = 624
TAIL = N - NS * RQ
BN = 1000

_mesh = plsc.VectorSubcoreMesh(core_axis_name="c", subcore_axis_name="s")


def _scale_chunk(buf, wgi, j):
    @pl.loop(0, CW // 16)
    def _(g):
        w16 = wgi[j, pl.ds(g * 16, 16)]
        for l in range(16):
            wv = w16[l]
            for jj in range(8):
                sl = (g * 16 + l, pl.ds(16 * jj, 16))
                buf[sl] = buf[sl] * wv


def _run_pass(h_hbm, acc, src_hbm, dst_hbm, w_hbm, cbase, ncht, off,
              srcg, dstg, wg, rows, semg, semsc):
    def gather(rowj, b):
        pltpu.async_copy(h_hbm.at[srcg.at[rowj]], rows[b], semg).wait()

    def scatter(rowj, b):
        pltpu.async_copy(rows[b], acc.at[dstg.at[rowj]], semsc[b],
                         add=True).wait()

    def load_group(row0):
        sl = pl.ds(pl.multiple_of(row0, GC), GC)
        pltpu.sync_copy(src_hbm.at[sl], srcg)
        pltpu.sync_copy(dst_hbm.at[sl], dstg)
        pltpu.sync_copy(w_hbm.at[sl], wg)
        if off is not None:
            for rj in range(GC):
                for i in range(CW // 16):
                    sli = (rj, pl.ds(16 * i, 16))
                    srcg[sli] = srcg[sli] + off

    @pl.loop(0, ncht, step=2)
    def _(k0):
        jg = k0 % GC

        @pl.when(jg == 0)
        def _():
            load_group(cbase + k0)

        for u in range(2):
            j = jg + u
            gather(j, u)
            _scale_chunk(rows[u], wg, j)
            scatter(j, u)


def _zero_acc(z_hbm, acc, s):
    r0 = s * RQ
    pltpu.sync_copy(z_hbm.at[pl.ds(r0, RQ)], acc.at[pl.ds(r0, RQ)])

    @pl.when(s == NS - 1)
    def _():
        pltpu.sync_copy(z_hbm.at[pl.ds(NS * RQ, TAIL)],
                        acc.at[pl.ds(NS * RQ, TAIL)])


def _sc_scratch():
    return ([pltpu.VMEM_SHARED((N, 128), jnp.float32)]
            + [pltpu.VMEM((GC, CW), jnp.int32) for _ in range(2)]
            + [pltpu.VMEM((GC, CW), jnp.float32)]
            + [pltpu.VMEM((CW, 128), jnp.float32) for _ in range(2)]
            + [pltpu.SemaphoreType.DMA for _ in range(3)])


def _split_bufs(bufs):
    srcg, dstg, wg = bufs[0], bufs[1], bufs[2]
    rows = list(bufs[3:5])
    semg = bufs[5]
    semsc = list(bufs[6:8])
    return srcg, dstg, wg, rows, semg, semsc


@functools.partial(
    pl.kernel,
    out_type=jax.ShapeDtypeStruct((NC, N, F_IN), jnp.float32),
    mesh=_mesh,
    scratch_types=_sc_scratch(),
)
def _sc_agg1(x_hbm, src_hbm, dst_hbm, w_hbm, z_hbm, out_hbm, acc, *bufs):
    srcg, dstg, wg, rows, semg, semsc = _split_bufs(bufs)
    c = lax.axis_index("c")
    s = lax.axis_index("s")
    cbase = (c * (NCH // NC // GC) + s * NG1) * GC
    _zero_acc(z_hbm, acc, s)
    plsc.subcore_barrier()

    _run_pass(x_hbm, acc, src_hbm, dst_hbm, w_hbm, cbase, NG1 * GC, None,
              srcg, dstg, wg, rows, semg, semsc)

    plsc.subcore_barrier()
    r0 = s * RQ
    pltpu.sync_copy(acc.at[pl.ds(r0, RQ)], out_hbm.at[c, pl.ds(r0, RQ)])

    @pl.when(s == NS - 1)
    def _():
        pltpu.sync_copy(acc.at[pl.ds(NS * RQ, TAIL)],
                        out_hbm.at[c, pl.ds(NS * RQ, TAIL)])


@functools.partial(
    pl.kernel,
    out_type=jax.ShapeDtypeStruct((N, H), jnp.float32),
    mesh=_mesh,
    scratch_types=_sc_scratch(),
)
def _sc_agg2(h_hbm, src_hbm, dst_hbm, w_hbm, z_hbm, out_hbm, acc, *bufs):
    srcg, dstg, wg, rows, semg, semsc = _split_bufs(bufs)
    c = lax.axis_index("c")
    s = lax.axis_index("s")
    cbase = s * NG2 * GC

    for p in range(2):
        cc = 2 * c + p
        _zero_acc(z_hbm, acc, s)
        plsc.subcore_barrier()

        _run_pass(h_hbm, acc, src_hbm, dst_hbm, w_hbm, cbase, NG2 * GC,
                  cc * N, srcg, dstg, wg, rows, semg, semsc)

        plsc.subcore_barrier()
        r0 = s * RQ
        pltpu.sync_copy(acc.at[pl.ds(r0, RQ)],
                        out_hbm.at[pl.ds(r0, RQ), pl.ds(cc * 128, 128)])

        @pl.when(s == NS - 1)
        def _():
            pltpu.sync_copy(acc.at[pl.ds(NS * RQ, TAIL)],
                            out_hbm.at[pl.ds(NS * RQ, TAIL),
                                       pl.ds(cc * 128, 128)])
        plsc.subcore_barrier()


def _tc1(p, x, W_rel1, b1, W_root1):
    def body(p_ref, x_ref, wr, bb, wo, o_ref):
        agg = p_ref[0] + p_ref[1]
        h = jnp.dot(agg, wr[...], preferred_element_type=jnp.float32, precision=lax.Precision.HIGHEST)
        h = h + bb[...] + jnp.dot(x_ref[...], wo[...],
                                  preferred_element_type=jnp.float32, precision=lax.Precision.HIGHEST)
        h = jnp.maximum(h, 0.0)
        for cc in range(4):
            o_ref[cc] = h[:, 128 * cc:128 * (cc + 1)]

    return pl.pallas_call(
        body,
        grid=(N // BN,),
        in_specs=[
            pl.BlockSpec((NC, BN, F_IN), lambda i: (0, i, 0)),
            pl.BlockSpec((BN, F_IN), lambda i: (i, 0)),
            pl.BlockSpec((F_IN, H), lambda i: (0, 0)),
            pl.BlockSpec((1, H), lambda i: (0, 0)),
            pl.BlockSpec((F_IN, H), lambda i: (0, 0)),
        ],
        out_specs=pl.BlockSpec((4, BN, 128), lambda i: (0, i, 0)),
        out_shape=jax.ShapeDtypeStruct((4, N, 128), jnp.float32),
    )(p, x, W_rel1, b1, W_root1)


def _tc2(agg2, h1c, batchf, W_rel2, b2, W_root2):
    def body(a_ref, h1_ref, bt_ref, wr, bb, wo, pool_ref):
        i = pl.program_id(0)
        h2 = jnp.dot(a_ref[...], wr[...],
                     preferred_element_type=jnp.float32, precision=lax.Precision.HIGHEST) + bb[...]
        for cc in range(4):
            h2 = h2 + jnp.dot(h1_ref[cc], wo[pl.ds(128 * cc, 128), :],
                              preferred_element_type=jnp.float32, precision=lax.Precision.HIGHEST)
        h2 = jnp.maximum(h2, 0.0)
        bt = bt_ref[...]
        gid = lax.broadcasted_iota(jnp.int32, (1, G), 1).astype(jnp.float32)
        onehot = (bt == gid).astype(jnp.float32)
        pool = lax.dot_general(onehot, h2, (((0,), (0,)), ((), ())),
                               preferred_element_type=jnp.float32, precision=lax.Precision.HIGHEST)

        @pl.when(i == 0)
        def _():
            pool_ref[...] = pool

        @pl.when(i > 0)
        def _():
            pool_ref[...] = pool_ref[...] + pool

    return pl.pallas_call(
        body,
        grid=(N // BN,),
        in_specs=[
            pl.BlockSpec((BN, H), lambda i: (i, 0)),
            pl.BlockSpec((4, BN, 128), lambda i: (0, i, 0)),
            pl.BlockSpec((BN, 1), lambda i: (i, 0)),
            pl.BlockSpec((H, H), lambda i: (0, 0)),
            pl.BlockSpec((1, H), lambda i: (0, 0)),
            pl.BlockSpec((H, H), lambda i: (0, 0)),
        ],
        out_specs=pl.BlockSpec((G, H), lambda i: (0, 0)),
        out_shape=jax.ShapeDtypeStruct((G, H), jnp.float32),
    )(agg2, h1c, batchf, W_rel2, b2, W_root2)


def _tc3(pool, batchf, W_l1, b_l1, W_l2, b_l2, W_out, b_out):
    def body(pool_ref, bt_ref, w1, b1, w2, b2, w3, b3, o_ref):
        bt = bt_ref[...]
        gid = lax.broadcasted_iota(jnp.int32, (1, G), 1).astype(jnp.float32)
        onehot = (bt == gid).astype(jnp.float32)
        ones = jnp.ones((N, 1), jnp.float32)
        cnt = lax.dot_general(onehot, ones, (((0,), (0,)), ((), ())),
                              preferred_element_type=jnp.float32, precision=lax.Precision.HIGHEST)
        mean = pool_ref[...] / jnp.maximum(cnt, 1.0)
        z = jnp.maximum(jnp.dot(mean, w1[...],
                                preferred_element_type=jnp.float32, precision=lax.Precision.HIGHEST)
                        + b1[...], 0.0)
        z = jnp.dot(z, w2[...], preferred_element_type=jnp.float32, precision=lax.Precision.HIGHEST) + b2[...]
        z = jnp.maximum(z, 0.0)
        o_ref[...] = jnp.dot(z, w3[...],
                             preferred_element_type=jnp.float32, precision=lax.Precision.HIGHEST) + b3[...]

    return pl.pallas_call(
        body,
        out_shape=jax.ShapeDtypeStruct((G, 1), jnp.float32),
    )(pool, batchf, W_l1, b_l1, W_l2, b_l2, W_out, b_out)


def kernel(x, edge_index, edge_attr, batch,
           W_rel1, b_rel1, W_root1,
           W_rel2, b_rel2, W_root2,
           W_l1, b_l1, W_l2, b_l2, W_out, b_out):
    pad = EP - E
    src2 = jnp.concatenate(
        [edge_index[0], jnp.zeros((pad,), jnp.int32)]).reshape(NCH, CW)
    dst2 = jnp.concatenate(
        [edge_index[1], jnp.zeros((pad,), jnp.int32)]).reshape(NCH, CW)
    w2 = jnp.concatenate(
        [edge_attr, jnp.zeros((pad,), jnp.float32)]).reshape(NCH, CW)
    zeros = jnp.zeros((N, 128), jnp.float32)
    batchf = batch.astype(jnp.float32).reshape(N, 1)

    p = _sc_agg1(x, src2, dst2, w2, zeros)
    h1c = _tc1(p, x, W_rel1, b_rel1.reshape(1, H), W_root1)
    agg2 = _sc_agg2(h1c.reshape(4 * N, 128), src2, dst2, w2, zeros)
    pool = _tc2(agg2, h1c, batchf, W_rel2, b_rel2.reshape(1, H), W_root2)
    out = _tc3(pool, batchf, W_l1, b_l1.reshape(1, G), W_l2,
               b_l2.reshape(1, 16), W_out, b_out.reshape(1, 1))
    return out


# speedup vs baseline: 1.3495x; 1.3495x over previous
import functools

import jax
import jax.numpy as jnp
from jax import lax
from jax.experimental import pallas as pl
from jax.experimental.pallas import tpu as pltpu
from jax.experimental.pallas import tpu_sc as plsc

N = 10000
E = 320000
F_IN = 128
H = 512
G = 64

NC = 2
NS = 16
CW = 128
NCH = E // CW
RQ = 624
TAIL = N - NS * RQ
BN = 1000
PREC = lax.Precision.HIGHEST

_mesh = plsc.VectorSubcoreMesh(core_axis_name="c", subcore_axis_name="s")


def _scale_rows(rows, wb, ncols16):
    @pl.loop(0, CW // 16)
    def _(g):
        w16 = wb[0, pl.ds(g * 16, 16)]
        for l in range(16):
            wv = w16[l]
            e = g * 16 + l
            for j in range(ncols16):
                sl = (e, pl.ds(16 * j, 16))
                rows[sl] = rows[sl] * wv


def _zero_acc(z_hbm, acc, s):
    r0 = s * RQ
    pltpu.sync_copy(z_hbm.at[pl.ds(r0, RQ)], acc.at[pl.ds(r0, RQ)])

    @pl.when(s == NS - 1)
    def _():
        pltpu.sync_copy(z_hbm.at[pl.ds(NS * RQ, TAIL)],
                        acc.at[pl.ds(NS * RQ, TAIL)])


@functools.partial(
    pl.kernel,
    out_type=jax.ShapeDtypeStruct((NC, N, F_IN), jnp.float32),
    mesh=_mesh,
    scratch_types=[
        pltpu.VMEM_SHARED((N, F_IN), jnp.float32),
        pltpu.VMEM((1, CW), jnp.int32),
        pltpu.VMEM((1, CW), jnp.int32),
        pltpu.VMEM((1, CW), jnp.float32),
        pltpu.VMEM((CW, F_IN), jnp.float32),
        pltpu.SemaphoreType.DMA,
    ],
)
def _sc_agg1(x_hbm, src_hbm, dst_hbm, w_hbm, z_hbm, out_hbm,
             acc, srcb, dstb, wb, rows, sem):
    c = lax.axis_index("c")
    s = lax.axis_index("s")
    r0 = s * RQ
    _zero_acc(z_hbm, acc, s)
    plsc.subcore_barrier()

    per_core = NCH // NC
    q, r = per_core // NS, per_core % NS
    base = c * per_core + s * q + jnp.minimum(s, r)
    nch = q + jnp.where(s < r, 1, 0)

    @pl.loop(0, q + 1)
    def _(t):
        @pl.when(t < nch)
        def _():
            ci = base + t
            pltpu.sync_copy(src_hbm.at[ci], srcb)
            pltpu.sync_copy(dst_hbm.at[ci], dstb)
            pltpu.sync_copy(w_hbm.at[ci], wb)
            pltpu.async_copy(x_hbm.at[srcb.at[0]], rows, sem).wait()
            _scale_rows(rows, wb, F_IN // 16)
            pltpu.sync_copy(rows, acc.at[dstb.at[0]], add=True)

    plsc.subcore_barrier()
    pltpu.sync_copy(acc.at[pl.ds(r0, RQ)], out_hbm.at[c, pl.ds(r0, RQ)])

    @pl.when(s == NS - 1)
    def _():
        pltpu.sync_copy(acc.at[pl.ds(NS * RQ, TAIL)],
                        out_hbm.at[c, pl.ds(NS * RQ, TAIL)])


@functools.partial(
    pl.kernel,
    out_type=jax.ShapeDtypeStruct((N, H), jnp.float32),
    mesh=_mesh,
    scratch_types=[
        pltpu.VMEM_SHARED((N, 128), jnp.float32),
        pltpu.VMEM((1, CW), jnp.int32),
        pltpu.VMEM((1, CW), jnp.int32),
        pltpu.VMEM((1, CW), jnp.float32),
        pltpu.VMEM((CW, 128), jnp.float32),
        pltpu.SemaphoreType.DMA,
    ],
)
def _sc_agg2(h_hbm, src_hbm, dst_hbm, w_hbm, z_hbm, out_hbm,
             acc, srcb, dstb, wb, rows, sem):
    c = lax.axis_index("c")
    s = lax.axis_index("s")
    r0 = s * RQ
    q, r = NCH // NS, NCH % NS
    base = s * q + jnp.minimum(s, r)
    nch = q + jnp.where(s < r, 1, 0)

    for p in range(2):
        cc = 2 * c + p
        off = cc * N
        _zero_acc(z_hbm, acc, s)
        plsc.subcore_barrier()

        @pl.loop(0, q + 1)
        def _(t):
            @pl.when(t < nch)
            def _():
                ci = base + t
                pltpu.sync_copy(src_hbm.at[ci], srcb)
                pltpu.sync_copy(dst_hbm.at[ci], dstb)
                pltpu.sync_copy(w_hbm.at[ci], wb)
                for i in range(CW // 16):
                    sl = (0, pl.ds(16 * i, 16))
                    srcb[sl] = srcb[sl] + off
                pltpu.async_copy(h_hbm.at[srcb.at[0]], rows, sem).wait()
                _scale_rows(rows, wb, 8)
                pltpu.sync_copy(rows, acc.at[dstb.at[0]], add=True)

        plsc.subcore_barrier()
        pltpu.sync_copy(acc.at[pl.ds(r0, RQ)],
                        out_hbm.at[pl.ds(r0, RQ), pl.ds(cc * 128, 128)])

        @pl.when(s == NS - 1)
        def _():
            pltpu.sync_copy(acc.at[pl.ds(NS * RQ, TAIL)],
                            out_hbm.at[pl.ds(NS * RQ, TAIL),
                                       pl.ds(cc * 128, 128)])
        plsc.subcore_barrier()


def _tc1(p, x, W_rel1, b1, W_root1):
    def body(p_ref, x_ref, wr, bb, wo, o_ref):
        agg = p_ref[0] + p_ref[1]
        h = jnp.dot(agg, wr[...], preferred_element_type=jnp.float32,
                    precision=PREC)
        h = h + bb[...] + jnp.dot(x_ref[...], wo[...],
                                  preferred_element_type=jnp.float32,
                                  precision=PREC)
        h = jnp.maximum(h, 0.0)
        for cc in range(4):
            o_ref[cc] = h[:, 128 * cc:128 * (cc + 1)]

    return pl.pallas_call(
        body,
        grid=(N // BN,),
        in_specs=[
            pl.BlockSpec((NC, BN, F_IN), lambda i: (0, i, 0)),
            pl.BlockSpec((BN, F_IN), lambda i: (i, 0)),
            pl.BlockSpec((F_IN, H), lambda i: (0, 0)),
            pl.BlockSpec((1, H), lambda i: (0, 0)),
            pl.BlockSpec((F_IN, H), lambda i: (0, 0)),
        ],
        out_specs=pl.BlockSpec((4, BN, 128), lambda i: (0, i, 0)),
        out_shape=jax.ShapeDtypeStruct((4, N, 128), jnp.float32),
    )(p, x, W_rel1, b1, W_root1)


def _tc2(agg2, h1c, batchf, W_rel2, b2, W_root2):
    def body(a_ref, h1_ref, bt_ref, wr, bb, wo, pool_ref):
        i = pl.program_id(0)
        h2 = jnp.dot(a_ref[...], wr[...],
                     preferred_element_type=jnp.float32,
                     precision=PREC) + bb[...]
        for cc in range(4):
            h2 = h2 + jnp.dot(h1_ref[cc], wo[pl.ds(128 * cc, 128), :],
                              preferred_element_type=jnp.float32,
                              precision=PREC)
        h2 = jnp.maximum(h2, 0.0)
        bt = bt_ref[...]
        gid = lax.broadcasted_iota(jnp.int32, (1, G), 1).astype(jnp.float32)
        onehot = (bt == gid).astype(jnp.float32)
        pool = lax.dot_general(onehot, h2, (((0,), (0,)), ((), ())),
                               preferred_element_type=jnp.float32,
                               precision=PREC)

        @pl.when(i == 0)
        def _():
            pool_ref[...] = pool

        @pl.when(i > 0)
        def _():
            pool_ref[...] = pool_ref[...] + pool

    return pl.pallas_call(
        body,
        grid=(N // BN,),
        in_specs=[
            pl.BlockSpec((BN, H), lambda i: (i, 0)),
            pl.BlockSpec((4, BN, 128), lambda i: (0, i, 0)),
            pl.BlockSpec((BN, 1), lambda i: (i, 0)),
            pl.BlockSpec((H, H), lambda i: (0, 0)),
            pl.BlockSpec((1, H), lambda i: (0, 0)),
            pl.BlockSpec((H, H), lambda i: (0, 0)),
        ],
        out_specs=pl.BlockSpec((G, H), lambda i: (0, 0)),
        out_shape=jax.ShapeDtypeStruct((G, H), jnp.float32),
    )(agg2, h1c, batchf, W_rel2, b2, W_root2)


def _tc3(pool, batchf, W_l1, b_l1, W_l2, b_l2, W_out, b_out):
    def body(pool_ref, bt_ref, w1, b1, w2, b2, w3, b3, o_ref):
        bt = bt_ref[...]
        gid = lax.broadcasted_iota(jnp.int32, (1, G), 1).astype(jnp.float32)
        onehot = (bt == gid).astype(jnp.float32)
        ones = jnp.ones((N, 1), jnp.float32)
        cnt = lax.dot_general(onehot, ones, (((0,), (0,)), ((), ())),
                              preferred_element_type=jnp.float32,
                              precision=PREC)
        mean = pool_ref[...] / jnp.maximum(cnt, 1.0)
        z = jnp.maximum(jnp.dot(mean, w1[...],
                                preferred_element_type=jnp.float32,
                                precision=PREC) + b1[...], 0.0)
        z = jnp.dot(z, w2[...], preferred_element_type=jnp.float32,
                    precision=PREC) + b2[...]
        z = jnp.maximum(z, 0.0)
        o_ref[...] = jnp.dot(z, w3[...], preferred_element_type=jnp.float32,
                             precision=PREC) + b3[...]

    return pl.pallas_call(
        body,
        out_shape=jax.ShapeDtypeStruct((G, 1), jnp.float32),
    )(pool, batchf, W_l1, b_l1, W_l2, b_l2, W_out, b_out)


def kernel(x, edge_index, edge_attr, batch,
           W_rel1, b_rel1, W_root1,
           W_rel2, b_rel2, W_root2,
           W_l1, b_l1, W_l2, b_l2, W_out, b_out):
    src2 = edge_index[0].reshape(NCH, 1, CW)
    dst2 = edge_index[1].reshape(NCH, 1, CW)
    w2 = edge_attr.reshape(NCH, 1, CW)
    zeros = jnp.zeros((N, 128), jnp.float32)
    batchf = batch.astype(jnp.float32).reshape(N, 1)

    p = _sc_agg1(x, src2, dst2, w2, zeros)
    h1c = _tc1(p, x, W_rel1, b_rel1.reshape(1, H), W_root1)
    agg2 = _sc_agg2(h1c.reshape(4 * N, 128), src2, dst2, w2, zeros)
    pool = _tc2(agg2, h1c, batchf, W_rel2, b_rel2.reshape(1, H), W_root2)
    out = _tc3(pool, batchf, W_l1, b_l1.reshape(1, G), W_l2,
               b_l2.reshape(1, 16), W_out, b_out.reshape(1, 1))
    return out


# speedup vs baseline: 1.6652x; 1.2340x over previous
import dataclasses
import functools

import jax
import jax.numpy as jnp
from jax import lax
from jax.experimental import pallas as pl
from jax.experimental.pallas import tpu as pltpu
from jax.experimental.pallas import tpu_sc as plsc

N = 10000
E = 320000
F_IN = 128
H = 512
G = 64

NC = 2
NS = 16
CW = 128
NCH = E // CW
RQ = 624
TAIL = N - NS * RQ
BN = 1000
PREC = lax.Precision.HIGHEST

_sc_params = pltpu.CompilerParams()
if "needs_layout_passes" in pltpu.CompilerParams.__dataclass_fields__:
    _sc_params = dataclasses.replace(_sc_params, needs_layout_passes=False)

_mesh = plsc.VectorSubcoreMesh(core_axis_name="c", subcore_axis_name="s")


def _scale_rows(rows, eb, ncols16):
    @pl.loop(0, CW // 16)
    def _(g):
        w16 = plsc.bitcast(eb[2, pl.ds(g * 16, 16)], jnp.float32)
        for l in range(16):
            wv = w16[l]
            e = g * 16 + l
            for j in range(ncols16):
                sl = (e, pl.ds(16 * j, 16))
                rows[sl] = rows[sl] * wv


def _zero_acc(z_hbm, acc, s):
    r0 = s * RQ
    pltpu.sync_copy(z_hbm.at[pl.ds(r0, RQ)], acc.at[pl.ds(r0, RQ)])

    @pl.when(s == NS - 1)
    def _():
        pltpu.sync_copy(z_hbm.at[pl.ds(NS * RQ, TAIL)],
                        acc.at[pl.ds(NS * RQ, TAIL)])


@functools.partial(
    pl.kernel,
    out_type=jax.ShapeDtypeStruct((NC, N, F_IN), jnp.float32),
    mesh=_mesh,
    scratch_types=[
        pltpu.VMEM_SHARED((N, F_IN), jnp.float32),
        pltpu.VMEM((3, CW), jnp.int32),
        pltpu.VMEM((CW, F_IN), jnp.float32),
        pltpu.SemaphoreType.DMA,
    ],
    compiler_params=_sc_params,
)
def _sc_agg1(x_hbm, e_hbm, z_hbm, out_hbm, acc, eb, rows, sem):
    c = lax.axis_index("c")
    s = lax.axis_index("s")
    r0 = s * RQ
    _zero_acc(z_hbm, acc, s)
    plsc.subcore_barrier()

    per_core = NCH // NC
    q, r = per_core // NS, per_core % NS
    base = c * per_core + s * q + jnp.minimum(s, r)
    nch = q + jnp.where(s < r, 1, 0)

    @pl.loop(0, q + 1)
    def _(t):
        @pl.when(t < nch)
        def _():
            ci = base + t
            pltpu.sync_copy(e_hbm.at[ci], eb)
            pltpu.async_copy(x_hbm.at[eb.at[0]], rows, sem).wait()
            _scale_rows(rows, eb, F_IN // 16)
            pltpu.sync_copy(rows, acc.at[eb.at[1]], add=True)

    plsc.subcore_barrier()
    pltpu.sync_copy(acc.at[pl.ds(r0, RQ)], out_hbm.at[c, pl.ds(r0, RQ)])

    @pl.when(s == NS - 1)
    def _():
        pltpu.sync_copy(acc.at[pl.ds(NS * RQ, TAIL)],
                        out_hbm.at[c, pl.ds(NS * RQ, TAIL)])


@functools.partial(
    pl.kernel,
    out_type=jax.ShapeDtypeStruct((N, H), jnp.float32),
    mesh=_mesh,
    scratch_types=[
        pltpu.VMEM_SHARED((N, 128), jnp.float32),
        pltpu.VMEM((3, CW), jnp.int32),
        pltpu.VMEM((CW, 128), jnp.float32),
        pltpu.SemaphoreType.DMA,
    ],
    compiler_params=_sc_params,
)
def _sc_agg2(h_hbm, e_hbm, z_hbm, out_hbm, acc, eb, rows, sem):
    c = lax.axis_index("c")
    s = lax.axis_index("s")
    r0 = s * RQ
    q, r = NCH // NS, NCH % NS
    base = s * q + jnp.minimum(s, r)
    nch = q + jnp.where(s < r, 1, 0)

    for p in range(2):
        cc = 2 * c + p
        off = cc * N
        _zero_acc(z_hbm, acc, s)
        plsc.subcore_barrier()

        @pl.loop(0, q + 1)
        def _(t):
            @pl.when(t < nch)
            def _():
                ci = base + t
                pltpu.sync_copy(e_hbm.at[ci], eb)
                for i in range(CW // 16):
                    sl = (0, pl.ds(16 * i, 16))
                    eb[sl] = eb[sl] + off
                pltpu.async_copy(h_hbm.at[eb.at[0]], rows, sem).wait()
                _scale_rows(rows, eb, 8)
                pltpu.sync_copy(rows, acc.at[eb.at[1]], add=True)

        plsc.subcore_barrier()
        pltpu.sync_copy(acc.at[pl.ds(r0, RQ)],
                        out_hbm.at[pl.ds(r0, RQ), pl.ds(cc * 128, 128)])

        @pl.when(s == NS - 1)
        def _():
            pltpu.sync_copy(acc.at[pl.ds(NS * RQ, TAIL)],
                            out_hbm.at[pl.ds(NS * RQ, TAIL),
                                       pl.ds(cc * 128, 128)])
        plsc.subcore_barrier()


def _tc1(p, x, W_rel1, b1, W_root1):
    def body(p_ref, x_ref, wr, bb, wo, o_ref):
        agg = p_ref[0] + p_ref[1]
        h = jnp.dot(agg, wr[...], preferred_element_type=jnp.float32,
                    precision=PREC)
        h = h + bb[...] + jnp.dot(x_ref[...], wo[...],
                                  preferred_element_type=jnp.float32,
                                  precision=PREC)
        h = jnp.maximum(h, 0.0)
        for cc in range(4):
            o_ref[cc] = h[:, 128 * cc:128 * (cc + 1)]

    return pl.pallas_call(
        body,
        grid=(N // BN,),
        in_specs=[
            pl.BlockSpec((NC, BN, F_IN), lambda i: (0, i, 0)),
            pl.BlockSpec((BN, F_IN), lambda i: (i, 0)),
            pl.BlockSpec((F_IN, H), lambda i: (0, 0)),
            pl.BlockSpec((1, H), lambda i: (0, 0)),
            pl.BlockSpec((F_IN, H), lambda i: (0, 0)),
        ],
        out_specs=pl.BlockSpec((4, BN, 128), lambda i: (0, i, 0)),
        out_shape=jax.ShapeDtypeStruct((4, N, 128), jnp.float32),
    )(p, x, W_rel1, b1, W_root1)


def _tc2(agg2, h1c, batchf, W_rel2, b2, W_root2):
    def body(a_ref, h1_ref, bt_ref, wr, bb, wo, pool_ref):
        i = pl.program_id(0)
        h2 = jnp.dot(a_ref[...], wr[...],
                     preferred_element_type=jnp.float32,
                     precision=PREC) + bb[...]
        for cc in range(4):
            h2 = h2 + jnp.dot(h1_ref[cc], wo[pl.ds(128 * cc, 128), :],
                              preferred_element_type=jnp.float32,
                              precision=PREC)
        h2 = jnp.maximum(h2, 0.0)
        bt = bt_ref[...]
        gid = lax.broadcasted_iota(jnp.int32, (1, G), 1).astype(jnp.float32)
        onehot = (bt == gid).astype(jnp.float32)
        pool = lax.dot_general(onehot, h2, (((0,), (0,)), ((), ())),
                               preferred_element_type=jnp.float32,
                               precision=PREC)

        @pl.when(i == 0)
        def _():
            pool_ref[...] = pool

        @pl.when(i > 0)
        def _():
            pool_ref[...] = pool_ref[...] + pool

    return pl.pallas_call(
        body,
        grid=(N // BN,),
        in_specs=[
            pl.BlockSpec((BN, H), lambda i: (i, 0)),
            pl.BlockSpec((4, BN, 128), lambda i: (0, i, 0)),
            pl.BlockSpec((BN, 1), lambda i: (i, 0)),
            pl.BlockSpec((H, H), lambda i: (0, 0)),
            pl.BlockSpec((1, H), lambda i: (0, 0)),
            pl.BlockSpec((H, H), lambda i: (0, 0)),
        ],
        out_specs=pl.BlockSpec((G, H), lambda i: (0, 0)),
        out_shape=jax.ShapeDtypeStruct((G, H), jnp.float32),
    )(agg2, h1c, batchf, W_rel2, b2, W_root2)


def _tc3(pool, batchf, W_l1, b_l1, W_l2, b_l2, W_out, b_out):
    def body(pool_ref, bt_ref, w1, b1, w2, b2, w3, b3, o_ref):
        bt = bt_ref[...]
        gid = lax.broadcasted_iota(jnp.int32, (1, G), 1).astype(jnp.float32)
        onehot = (bt == gid).astype(jnp.float32)
        ones = jnp.ones((N, 1), jnp.float32)
        cnt = lax.dot_general(onehot, ones, (((0,), (0,)), ((), ())),
                              preferred_element_type=jnp.float32,
                              precision=PREC)
        mean = pool_ref[...] / jnp.maximum(cnt, 1.0)
        z = jnp.maximum(jnp.dot(mean, w1[...],
                                preferred_element_type=jnp.float32,
                                precision=PREC) + b1[...], 0.0)
        z = jnp.dot(z, w2[...], preferred_element_type=jnp.float32,
                    precision=PREC) + b2[...]
        z = jnp.maximum(z, 0.0)
        o_ref[...] = jnp.dot(z, w3[...], preferred_element_type=jnp.float32,
                             precision=PREC) + b3[...]

    return pl.pallas_call(
        body,
        out_shape=jax.ShapeDtypeStruct((G, 1), jnp.float32),
    )(pool, batchf, W_l1, b_l1, W_l2, b_l2, W_out, b_out)


def kernel(x, edge_index, edge_attr, batch,
           W_rel1, b_rel1, W_root1,
           W_rel2, b_rel2, W_root2,
           W_l1, b_l1, W_l2, b_l2, W_out, b_out):
    wbits = lax.bitcast_convert_type(edge_attr, jnp.int32)
    edata = jnp.stack([edge_index[0].reshape(NCH, CW),
                       edge_index[1].reshape(NCH, CW),
                       wbits.reshape(NCH, CW)], axis=1)
    zeros = jnp.zeros((N, 128), jnp.float32)
    batchf = batch.astype(jnp.float32).reshape(N, 1)

    p = _sc_agg1(x, edata, zeros)
    h1c = _tc1(p, x, W_rel1, b_rel1.reshape(1, H), W_root1)
    agg2 = _sc_agg2(h1c.reshape(4 * N, 128), edata, zeros)
    pool = _tc2(agg2, h1c, batchf, W_rel2, b_rel2.reshape(1, H), W_root2)
    out = _tc3(pool, batchf, W_l1, b_l1.reshape(1, G), W_l2,
               b_l2.reshape(1, 16), W_out, b_out.reshape(1, 1))
    return out
